# baseline (speedup 1.0000x reference)
import functools

import jax
import jax.numpy as jnp
from jax import lax
from jax.experimental import pallas as pl
from jax.experimental.pallas import tpu as pltpu
from jax.experimental.pallas import tpu_sc as plsc

_OUT_DIM = 5
_DP = 8
_ETA_MIN = 0.8
_ETA_RANGE = 0.99 - 0.8

_B, _T = 16384, 200
_N = _B * _T
_LANES = 128
_ROWS = _N // _LANES
_NC, _NS = 2, 16
_NW = _NC * _NS
_RPW = _ROWS // _NW
_CH = 16
_NCH = _RPW // _CH
_NROW = _CH * _LANES
_V = 100000


def _sigmoid_body(x_ref, o_ref):
    x = x_ref[...]
    o_ref[...] = _ETA_MIN + _ETA_RANGE / (1.0 + jnp.exp(-x))


_transform = pl.pallas_call(
    _sigmoid_body,
    out_shape=jax.ShapeDtypeStruct((_V * _DP // _LANES, _LANES), jnp.float32),
)

_sc_mesh = plsc.VectorSubcoreMesh(core_axis_name="c", subcore_axis_name="s")


@functools.partial(
    pl.kernel,
    mesh=_sc_mesh,
    out_type=jax.ShapeDtypeStruct((_N, _OUT_DIM), jnp.float32),
    scratch_types=[
        pltpu.VMEM((_CH, _LANES), jnp.int32),
        pltpu.VMEM((_NROW, _DP), jnp.float32),
        pltpu.SemaphoreType.DMA,
    ],
    compiler_params=pltpu.CompilerParams(use_tc_tiling_on_sc=False),
)
def _gather_kernel(table_hbm, idx_hbm, out_hbm, idx_v, rows_v, sem):
    wid = lax.axis_index("s") * _NC + lax.axis_index("c")

    def chunk(k, carry):
        r0 = wid * _RPW + k * _CH
        pltpu.sync_copy(idx_hbm.at[pl.ds(r0, _CH)], idx_v)
        copies = [
            pltpu.async_copy(table_hbm.at[idx_v.at[j]],
                             rows_v.at[pl.ds(j * _LANES, _LANES)], sem)
            for j in range(_CH)
        ]
        for c in copies:
            c.wait()
        pltpu.sync_copy(rows_v.at[:, pl.ds(0, _OUT_DIM)],
                        out_hbm.at[pl.ds(r0 * _LANES, _NROW)])
        return carry

    lax.fori_loop(0, _NCH, chunk, 0)


def kernel(ops_t, cond_ids, eta_table):
    del ops_t
    padded = jnp.pad(eta_table, ((0, 0), (0, _DP - _OUT_DIM)))
    table = _transform(padded.reshape(-1, _LANES)).reshape(_V, _DP)
    idx = cond_ids.reshape(_ROWS, _LANES)
    out = _gather_kernel(table, idx)
    return out.reshape(_B, _T, _OUT_DIM)


# speedup vs baseline: 1.0120x; 1.0120x over previous
import functools

import jax
import jax.numpy as jnp
from jax import lax
from jax.experimental import pallas as pl
from jax.experimental.pallas import tpu as pltpu
from jax.experimental.pallas import tpu_sc as plsc

_OUT_DIM = 5
_DP = 8
_ETA_MIN = 0.8
_ETA_RANGE = 0.99 - 0.8

_B, _T = 16384, 200
_N = _B * _T
_LANES = 128
_ROWS = _N // _LANES
_NC, _NS = 2, 16
_NW = _NC * _NS
_RPW = _ROWS // _NW
_CH = 16
_NCH = _RPW // _CH
_NROW = _CH * _LANES
_V = 100000


def _sigmoid_body(x_ref, o_ref):
    x = x_ref[...]
    o_ref[...] = _ETA_MIN + _ETA_RANGE / (1.0 + jnp.exp(-x))


_transform = pl.pallas_call(
    _sigmoid_body,
    out_shape=jax.ShapeDtypeStruct((_V * _DP // _LANES, _LANES), jnp.float32),
)

_sc_mesh = plsc.VectorSubcoreMesh(core_axis_name="c", subcore_axis_name="s")


@functools.partial(
    pl.kernel,
    mesh=_sc_mesh,
    out_type=jax.ShapeDtypeStruct((_N, _OUT_DIM), jnp.float32),
    scratch_types=[
        pltpu.VMEM((_CH, _LANES), jnp.int32),
        pltpu.VMEM((_NROW, _DP), jnp.float32),
        pltpu.VMEM_SHARED((_V, _DP), jnp.float32),
        pltpu.SemaphoreType.DMA,
    ],
    compiler_params=pltpu.CompilerParams(use_tc_tiling_on_sc=False),
)
def _gather_kernel(table_hbm, idx_hbm, out_hbm, idx_v, rows_v, table_sp, sem):
    sid = lax.axis_index("s")
    wid = sid * _NC + lax.axis_index("c")

    @pl.when(sid == 0)
    def _stage():
        pltpu.sync_copy(table_hbm, table_sp)

    plsc.subcore_barrier()

    def chunk(k, carry):
        r0 = wid * _RPW + k * _CH
        pltpu.sync_copy(idx_hbm.at[pl.ds(r0, _CH)], idx_v)
        copies = [
            pltpu.async_copy(table_sp.at[idx_v.at[j]],
                             rows_v.at[pl.ds(j * _LANES, _LANES)], sem)
            for j in range(_CH)
        ]
        for c in copies:
            c.wait()
        pltpu.sync_copy(rows_v.at[:, pl.ds(0, _OUT_DIM)],
                        out_hbm.at[pl.ds(r0 * _LANES, _NROW)])
        return carry

    lax.fori_loop(0, _NCH, chunk, 0)


def kernel(ops_t, cond_ids, eta_table):
    del ops_t
    padded = jnp.pad(eta_table, ((0, 0), (0, _DP - _OUT_DIM)))
    table = _transform(padded.reshape(-1, _LANES)).reshape(_V, _DP)
    idx = cond_ids.reshape(_ROWS, _LANES)
    out = _gather_kernel(table, idx)
    return out.reshape(_B, _T, _OUT_DIM)


# speedup vs baseline: 1.0141x; 1.0020x over previous
import functools

import jax
import jax.numpy as jnp
from jax import lax
from jax.experimental import pallas as pl
from jax.experimental.pallas import tpu as pltpu
from jax.experimental.pallas import tpu_sc as plsc

_OUT_DIM = 5
_ETA_MIN = 0.8
_ETA_RANGE = 0.99 - 0.8

_B, _T = 16384, 200
_N = _B * _T
_LANES = 128
_ROWS = _N // _LANES
_NC, _NS = 2, 16
_NW = _NC * _NS
_RPW = _ROWS // _NW
_CH = 16
_NCH = _RPW // _CH
_NROW = _CH * _LANES
_V = 100000
_DP = 8


def _sigmoid_body(x_ref, o_ref):
    x = x_ref[...]
    o_ref[...] = _ETA_MIN + _ETA_RANGE / (1.0 + jnp.exp(-x))


_transform = pl.pallas_call(
    _sigmoid_body,
    out_shape=jax.ShapeDtypeStruct((_V * _DP // _LANES, _LANES), jnp.float32),
)

_sc_mesh = plsc.VectorSubcoreMesh(core_axis_name="c", subcore_axis_name="s")


@functools.partial(
    pl.kernel,
    mesh=_sc_mesh,
    out_type=jax.ShapeDtypeStruct((_N, _OUT_DIM), jnp.float32),
    scratch_types=[
        pltpu.VMEM((_CH, _LANES), jnp.int32),
        pltpu.VMEM((_NROW, _DP), jnp.float32),
        pltpu.VMEM_SHARED((_V, _DP), jnp.float32),
        pltpu.SemaphoreType.DMA,
    ],
    compiler_params=pltpu.CompilerParams(use_tc_tiling_on_sc=False),
)
def _gather_kernel(table_hbm, idx_hbm, out_hbm, idx_v, rows_v, table_sp, sem):
    sid = lax.axis_index("s")
    wid = sid * _NC + lax.axis_index("c")

    @pl.when(sid == 0)
    def _stage():
        pltpu.sync_copy(table_hbm, table_sp)

    plsc.subcore_barrier()

    def chunk(k, carry):
        r0 = wid * _RPW + k * _CH
        pltpu.sync_copy(idx_hbm.at[pl.ds(r0, _CH)], idx_v)
        copies = [
            pltpu.async_copy(table_sp.at[idx_v.at[j]],
                             rows_v.at[pl.ds(j * _LANES, _LANES)], sem)
            for j in range(_CH)
        ]
        for c in copies:
            c.wait()
        pltpu.sync_copy(rows_v.at[:, pl.ds(0, _OUT_DIM)],
                        out_hbm.at[pl.ds(r0 * _LANES, _NROW)])
        return carry

    lax.fori_loop(0, _NCH, chunk, 0)


def kernel(ops_t, cond_ids, eta_table):
    del ops_t
    padded = jnp.pad(eta_table, ((0, 0), (0, _DP - _OUT_DIM)))
    table = _transform(padded.reshape(-1, _LANES)).reshape(_V, _DP)
    idx = cond_ids.reshape(_ROWS, _LANES)
    out = _gather_kernel(table, idx)
    return out.reshape(_B, _T, _OUT_DIM)
